# baseline (speedup 1.0000x reference)
import functools

import jax
import jax.numpy as jnp
from jax import lax
from jax.experimental import pallas as pl
from jax.experimental.pallas import tpu as pltpu
from jax.experimental.pallas import tpu_sc as plsc

NC, NS, L = 2, 16, 16
NW = NC * NS
B = 16384
D = 64
TBL = 1000000
SEG = 512
SPW = 62
NSEG = 1954
TAIL_START = (NSEG - 1) * SEG
TAILW = TBL - TAIL_START
CAP = 1008
RPW = B // NW
REG = 0.01
EPSILON = 0.5

_MESH = dict(core_axis_name="c", subcore_axis_name="s",
             num_cores=NC, num_subcores=NS)


def _iota16():
    return lax.broadcasted_iota(jnp.int32, (L,), 0)


def _sc_assemble(users, pos_items, neg_items, t_user, t_item,
                 tail_user, tail_item):
    mesh = plsc.VectorSubcoreMesh(**_MESH)

    @functools.partial(
        pl.kernel,
        out_type=[
            jax.ShapeDtypeStruct((B + 8, 128), jnp.float32),
            jax.ShapeDtypeStruct((B + 8, 128), jnp.float32),
            jax.ShapeDtypeStruct((B + 8, 128), jnp.float32),
        ],
        mesh=mesh,
        compiler_params=pltpu.CompilerParams(needs_layout_passes=False),
        scratch_types=[
            pltpu.VMEM((D, SEG), jnp.float32),
            pltpu.VMEM((D, SEG), jnp.float32),
            pltpu.VMEM((D, SEG), jnp.float32),
            pltpu.VMEM((CAP + L,), jnp.int32),
            pltpu.VMEM((CAP + L,), jnp.int32),
            pltpu.VMEM((CAP + L,), jnp.int32),
            pltpu.VMEM((CAP + L,), jnp.int32),
            pltpu.VMEM((64, 128), jnp.float32),
            pltpu.VMEM((64, 128), jnp.float32),
            pltpu.VMEM((2, 64), jnp.int32),
            pltpu.VMEM((512,), jnp.int32),
            pltpu.VMEM((64,), jnp.int32),
            pltpu.VMEM((64,), jnp.int32),
            pltpu.VMEM((D, TAILW), jnp.float32),
            pltpu.SemaphoreType.DMA,
            pltpu.SemaphoreType.DMA,
            pltpu.SemaphoreType.DMA,
            pltpu.SemaphoreType.DMA,
        ],
    )
    def sc_kernel(users_h, pos_h, neg_h, tu_h, ti_h, tlu_h, tli_h,
                  urows_h, prows_h, nrows_h,
                  b0, b1, b2, larA, ldsA, larB, ldsB, st0, st1, ixm,
                  scanbuf, prow, pdst, tailbuf, sem0, sem1, sem2, semsc):
        w = lax.axis_index("s") * NC + lax.axis_index("c")
        seg_lo = w * SPW
        ix0 = ixm.at[0]
        ix1 = ixm.at[1]
        ii = _iota16()
        zz = jnp.zeros((L,), jnp.int32)

        def seg_id(k):
            return seg_lo + k

        def stream_start(tbl_h, k, bufref, sem):
            s = seg_id(k)

            @pl.when((k < SPW) & (s < NSEG - 1))
            def _():
                off = pl.multiple_of(s * SEG, 128)
                pltpu.async_copy(tbl_h.at[:, pl.ds(off, SEG)], bufref, sem)

        def stream_wait(tbl_h, k, bufref, sem):
            s = seg_id(k)

            @pl.when((k < SPW) & (s < NSEG - 1))
            def _():
                pltpu.make_async_copy(tbl_h.at[:, pl.ds(0, SEG)],
                                      bufref, sem).wait()

        def fill(arr_h, lrows, ldest):
            def cbody(c, nl):
                pltpu.sync_copy(arr_h.at[pl.ds(c * 512, 512)], scanbuf)

                def vbody(v, nl):
                    rows = scanbuf[pl.ds(v * L, L)]
                    seg = lax.shift_right_logical(rows, 9)
                    m = (seg >= seg_lo) & (seg < seg_lo + SPW)
                    cnt = plsc.all_reduce_population_count(m)[0]
                    ok = nl <= CAP - L

                    @pl.when(ok & (cnt > 0))
                    def _():
                        dst = c * 512 + v * L + ii
                        pos = nl + plsc.cumsum(m.astype(jnp.int32)) - 1
                        plsc.store_scatter(lrows, [pos], rows, mask=m)
                        plsc.store_scatter(ldest, [pos], dst, mask=m)

                    return lax.select(ok, nl + cnt, nl)

                return lax.fori_loop(0, 512 // L, vbody, nl)

            return lax.fori_loop(0, B // 512, cbody, jnp.int32(0))

        def init_ix(ix):
            for g in range(64 // L):
                ix[pl.ds(g * L, L)] = jnp.full((L,), B, jnp.int32)

        def emit_chunk(base, m, seg_off, bufref, st, ix, out_h):
            r16 = prow[pl.ds(0, L)]
            d16 = pdst[pl.ds(0, L)]
            rl = r16 - seg_off
            pos = base + plsc.cumsum(m.astype(jnp.int32)) - 1
            plsc.store_scatter(ix, [pos], d16, mask=m)

            def jbody(jj, _):
                for sub in range(4):
                    js = jnp.full((L,), jj * 4 + sub, jnp.int32)
                    val = plsc.load_gather(bufref, [js, rl], mask=m)
                    plsc.store_scatter(st, [pos, js], val, mask=m)
                return 0

            lax.fori_loop(0, D // 4, jbody, 0)

        def flush(st, ix, out_h):
            pltpu.async_copy(st, out_h.at[ix], semsc).wait()

        def process_list(lrows, ldest, nlist, s, bufref, st, ix, out_h, pend,
                         tail=False):
            seg_off = lax.min(s * SEG, jnp.int32(TAIL_START))
            nv = lax.shift_right_logical(nlist + L - 1, 4)

            def vbody(v, carry):
                npend, pend = carry
                off = v * L
                rows = lrows[pl.ds(off, L)]
                valid = ii < (nlist - off)
                if tail:
                    m = valid & (rows >= TAIL_START)
                else:
                    m = (valid & (lax.shift_right_logical(rows, 9) == s)
                         & (rows < TAIL_START))
                cnt = plsc.all_reduce_population_count(m)[0]
                np2 = npend + cnt
                full = np2 >= L
                flush_now = full & (pend > 48)
                base = lax.select(flush_now, jnp.int32(0), pend)

                @pl.when(cnt > 0)
                def _():
                    dest = ldest[pl.ds(off, L)]
                    ppos = npend + plsc.cumsum(m.astype(jnp.int32)) - 1
                    plsc.store_scatter(prow, [ppos], rows, mask=m)
                    plsc.store_scatter(pdst, [ppos], dest, mask=m)

                    @pl.when(flush_now)
                    def _():
                        flush(st, ix, out_h)

                    @pl.when(full)
                    def _():
                        emit_chunk(base, ii >= 0, seg_off, bufref, st, ix,
                                   out_h)
                        l1 = prow[pl.ds(L, L)]
                        prow[pl.ds(0, L)] = l1
                        l2 = pdst[pl.ds(L, L)]
                        pdst[pl.ds(0, L)] = l2

                return (lax.select(full, np2 - L, np2),
                        lax.select(full, base + L, pend))

            npend, pend = lax.fori_loop(0, nv, vbody, (jnp.int32(0), pend))
            flush_now = (npend > 0) & (pend > 48)
            base = lax.select(flush_now, jnp.int32(0), pend)

            @pl.when(flush_now)
            def _():
                flush(st, ix, out_h)

            @pl.when(npend > 0)
            def _():
                emit_chunk(base, ii < npend, seg_off, bufref, st, ix, out_h)

            return lax.select(npend > 0, base + npend, pend)

        def phase(tbl_h, tail_h, lists):
            stream_start(tbl_h, 0, b0, sem0)
            stream_start(tbl_h, 1, b1, sem1)
            stream_start(tbl_h, 2, b2, sem2)

            def slot(k, bufref, sem, pends):
                stream_wait(tbl_h, k, bufref, sem)
                s = seg_id(k)
                new_pends = tuple(
                    process_list(lrows, ldest, nlist, s, bufref,
                                 st, ix, out_h, pend)
                    for (lrows, ldest, nlist, st, ix, out_h), pend
                    in zip(lists, pends))
                stream_start(tbl_h, k + 3, bufref, sem)
                return new_pends

            def triobody(k3, pends):
                k = 3 * k3
                pends = slot(k, b0, sem0, pends)
                pends = slot(k + 1, b1, sem1, pends)
                pends = slot(k + 2, b2, sem2, pends)
                return pends

            pends = lax.fori_loop(0, (SPW + 2) // 3, triobody,
                                  tuple(jnp.int32(0) for _ in lists))
            pltpu.sync_copy(tail_h, tailbuf)
            for (lrows, ldest, nlist, st, ix, out_h), pend in zip(lists, pends):
                process_list(lrows, ldest, nlist, jnp.int32(NSEG - 1), tailbuf,
                             st, ix, out_h, pend, tail=True)
                flush(st, ix, out_h)

        init_ix(ix0)
        init_ix(ix1)
        nl_u = fill(users_h, larA, ldsA)
        phase(tu_h, tlu_h, [(larA, ldsA, nl_u, st0, ix0, urows_h)])
        init_ix(ix0)
        init_ix(ix1)
        nl_p = fill(pos_h, larA, ldsA)
        nl_n = fill(neg_h, larB, ldsB)
        phase(ti_h, tli_h, [(larA, ldsA, nl_p, st0, ix0, prows_h),
                            (larB, ldsB, nl_n, st1, ix1, nrows_h)])

    return sc_kernel(users, pos_items, neg_items, t_user, t_item,
                     tail_user, tail_item)


BLK = 2048
NBLK = (B + 8 + BLK - 1) // BLK


def _tc_rows_body(u_ref, p_ref, n_ref, s_ref, uu_ref, dd_ref, pn_ref):
    colv = lax.broadcasted_iota(jnp.int32, (BLK, 128), 1) < D
    U = jnp.where(colv, u_ref[...], 0.0)
    P = jnp.where(colv, p_ref[...], 0.0)
    Nn = jnp.where(colv, n_ref[...], 0.0)
    dv = P - Nn
    s_ref[...] = jnp.sum(U * dv, axis=1).reshape(1, 1, BLK)
    uu_ref[...] = jnp.sum(U * U, axis=1).reshape(1, 1, BLK)
    dd_ref[...] = jnp.sum(dv * dv, axis=1).reshape(1, 1, BLK)
    pn_ref[...] = jnp.sum(P * P + Nn * Nn, axis=1).reshape(1, 1, BLK)


def _tc_rows(ur, pr, nr):
    blk_in = pl.BlockSpec((BLK, 128), lambda i: (i, 0))
    blk_out = pl.BlockSpec((1, 1, BLK), lambda i: (i, 0, 0))
    out = jax.ShapeDtypeStruct((NBLK, 1, BLK), jnp.float32)
    return pl.pallas_call(
        _tc_rows_body,
        grid=(NBLK,),
        in_specs=[blk_in, blk_in, blk_in],
        out_specs=[blk_out, blk_out, blk_out, blk_out],
        out_shape=[out, out, out, out],
    )(ur, pr, nr)


def _tc_final_body(s_ref, uu_ref, dd_ref, pn_ref, out_ref):
    rowv = (lax.broadcasted_iota(jnp.int32, (NBLK, BLK), 0) * BLK
            + lax.broadcasted_iota(jnp.int32, (NBLK, BLK), 1)) < B
    s = jnp.where(rowv, s_ref[...], 0.0)
    uu = jnp.where(rowv, uu_ref[...], 0.0)
    dd = jnp.where(rowv, dd_ref[...], 0.0)
    pn = jnp.where(rowv, pn_ref[...], 0.0)

    g = (-1.0 / B) / (1.0 + jnp.exp(s))
    gsq = g * g
    norm_u = jnp.sqrt(jnp.sum(gsq * dd))
    norm_i = jnp.sqrt(jnp.sum(gsq * uu))
    a = EPSILON / (norm_u + 1e-8)
    b = EPSILON / (norm_i + 1e-8)
    s_adv = s + 2.0 * b * g * uu + a * g * dd + 2.0 * a * b * gsq * s

    def logsig(x):
        return jnp.minimum(x, 0.0) - jnp.log1p(jnp.exp(-jnp.abs(x)))

    bpr = -jnp.sum(jnp.where(rowv, logsig(s), 0.0)) / B
    adv = -jnp.sum(jnp.where(rowv, logsig(s_adv), 0.0)) / B
    reg = REG * (jnp.sum(uu) + jnp.sum(pn))
    out_ref[0, 0] = bpr + adv + reg


def _tc_final(s, uu, dd, pn):
    return pl.pallas_call(
        _tc_final_body,
        out_shape=jax.ShapeDtypeStruct((1, 1), jnp.float32),
        out_specs=pl.BlockSpec(memory_space=pltpu.SMEM),
    )(s, uu, dd, pn)


def kernel(users, pos_items, neg_items, user_emb, item_emb):
    t_u = jnp.swapaxes(user_emb, 0, 1)
    t_i = jnp.swapaxes(item_emb, 0, 1)
    tail_u = lax.slice(t_u, (0, TAIL_START), (D, TBL))
    tail_i = lax.slice(t_i, (0, TAIL_START), (D, TBL))
    ur, pr, nr = _sc_assemble(users, pos_items, neg_items, t_u, t_i,
                              tail_u, tail_i)
    s, uu, dd, pn = _tc_rows(ur, pr, nr)
    out = _tc_final(s.reshape(NBLK, BLK), uu.reshape(NBLK, BLK),
                    dd.reshape(NBLK, BLK), pn.reshape(NBLK, BLK))
    return out[0, 0]
